# Initial kernel scaffold; baseline (speedup 1.0000x reference)
import jax
import jax.numpy as jnp
from jax.experimental import pallas as pl


def kernel(x, species_idx, shifts, scales):
    raise NotImplementedError("write your pallas kernel here")


# speedup vs baseline: 3.7421x; 3.7421x over previous
import functools

import jax
import jax.numpy as jnp
from jax import lax
from jax.experimental import pallas as pl
from jax.experimental.pallas import tpu as pltpu
from jax.experimental.pallas import tpu_sc as plsc

_N = 100000
_S = 64
_L = 16
_NC = 2
_NS = 16
_NW = _NC * _NS
_CHUNK = 3136
_LAST = _N - (_NW - 1) * _CHUNK


def _sc_body(x_hbm, idx_hbm, shifts_hbm, scales_hbm, out_hbm,
             idx_v, x_v, o_v, sh_v, sc_v):
    wid = lax.axis_index("s") * _NC + lax.axis_index("c")
    base = wid * _CHUNK

    pltpu.sync_copy(shifts_hbm, sh_v)
    pltpu.sync_copy(scales_hbm, sc_v)

    def run(n):
        pltpu.sync_copy(idx_hbm.at[pl.ds(base, n)], idx_v.at[pl.ds(0, n)])
        pltpu.sync_copy(x_hbm.at[pl.ds(base, n)], x_v.at[pl.ds(0, n)])

        def step(i, carry):
            o = i * _L
            iv = idx_v[pl.ds(o, _L)]
            xv = x_v[pl.ds(o, _L)]
            sh = plsc.load_gather(sh_v, [iv])
            sc = plsc.load_gather(sc_v, [iv])
            o_v[pl.ds(o, _L)] = sh + sc * xv
            return carry

        lax.fori_loop(0, n // _L, step, 0)
        pltpu.sync_copy(o_v.at[pl.ds(0, n)], out_hbm.at[pl.ds(base, n)])

    @pl.when(wid < _NW - 1)
    def _full():
        run(_CHUNK)

    @pl.when(wid == _NW - 1)
    def _tail():
        run(_LAST)


@jax.jit
def _sc_shift(x_flat, idx, shifts, scales):
    mesh = plsc.VectorSubcoreMesh(core_axis_name="c", subcore_axis_name="s")
    fn = pl.kernel(
        _sc_body,
        out_type=jax.ShapeDtypeStruct((_N,), jnp.float32),
        mesh=mesh,
        scratch_types=[
            pltpu.VMEM((_CHUNK,), jnp.int32),
            pltpu.VMEM((_CHUNK,), jnp.float32),
            pltpu.VMEM((_CHUNK,), jnp.float32),
            pltpu.VMEM((_S,), jnp.float32),
            pltpu.VMEM((_S,), jnp.float32),
        ],
        compiler_params=pltpu.CompilerParams(needs_layout_passes=False),
    )
    return fn(x_flat, idx, shifts, scales)


def kernel(x, species_idx, shifts, scales):
    out = _sc_shift(x.reshape(-1), species_idx.astype(jnp.int32),
                    shifts, scales)
    return out.reshape(_N, 1)


# speedup vs baseline: 3.8703x; 1.0343x over previous
import functools

import jax
import jax.numpy as jnp
from jax import lax
from jax.experimental import pallas as pl
from jax.experimental.pallas import tpu as pltpu
from jax.experimental.pallas import tpu_sc as plsc

_N = 100000
_S = 64
_L = 16
_NC = 2
_NS = 16
_NW = _NC * _NS
_CHUNK = 3136
_LAST = _N - (_NW - 1) * _CHUNK


def _sc_body(x_hbm, idx_hbm, shifts_hbm, scales_hbm, out_hbm,
             idx_v, x_v, o_v, sh_v, sc_v):
    wid = lax.axis_index("s") * _NC + lax.axis_index("c")
    base = wid * _CHUNK

    pltpu.sync_copy(shifts_hbm, sh_v)
    pltpu.sync_copy(scales_hbm, sc_v)

    def run(n):
        pltpu.sync_copy(idx_hbm.at[pl.ds(base, n)], idx_v.at[pl.ds(0, n)])
        pltpu.sync_copy(x_hbm.at[pl.ds(base, n)], x_v.at[pl.ds(0, n)])

        @plsc.parallel_loop(0, n, step=_L, unroll=8)
        def _step(o):
            iv = idx_v[pl.ds(o, _L)]
            xv = x_v[pl.ds(o, _L)]
            sh = plsc.load_gather(sh_v, [iv])
            sc = plsc.load_gather(sc_v, [iv])
            o_v[pl.ds(o, _L)] = sh + sc * xv
        pltpu.sync_copy(o_v.at[pl.ds(0, n)], out_hbm.at[pl.ds(base, n)])

    @pl.when(wid < _NW - 1)
    def _full():
        run(_CHUNK)

    @pl.when(wid == _NW - 1)
    def _tail():
        run(_LAST)


@jax.jit
def _sc_shift(x_flat, idx, shifts, scales):
    mesh = plsc.VectorSubcoreMesh(core_axis_name="c", subcore_axis_name="s")
    fn = pl.kernel(
        _sc_body,
        out_type=jax.ShapeDtypeStruct((_N,), jnp.float32),
        mesh=mesh,
        scratch_types=[
            pltpu.VMEM((_CHUNK,), jnp.int32),
            pltpu.VMEM((_CHUNK,), jnp.float32),
            pltpu.VMEM((_CHUNK,), jnp.float32),
            pltpu.VMEM((_S,), jnp.float32),
            pltpu.VMEM((_S,), jnp.float32),
        ],
        compiler_params=pltpu.CompilerParams(needs_layout_passes=False),
    )
    return fn(x_flat, idx, shifts, scales)


def kernel(x, species_idx, shifts, scales):
    out = _sc_shift(x.reshape(-1), species_idx.astype(jnp.int32),
                    shifts, scales)
    return out.reshape(_N, 1)


# speedup vs baseline: 4.1298x; 1.0671x over previous
import jax
import jax.numpy as jnp
from jax import lax
from jax.experimental import pallas as pl
from jax.experimental.pallas import tpu as pltpu
from jax.experimental.pallas import tpu_sc as plsc

_N = 100000
_S = 64
_L = 16
_NC = 2
_NS = 16
_NW = _NC * _NS
_CHUNK = 3136


def _sc_body(x_hbm, idx_hbm, shifts_hbm, scales_hbm, out_hbm,
             idx_v, x_v, o_v, sh_v, sc_v, sem):
    wid = lax.axis_index("s") * _NC + lax.axis_index("c")
    base = jnp.minimum(wid * _CHUNK, _N - _CHUNK)

    c1 = pltpu.async_copy(shifts_hbm, sh_v, sem)
    c2 = pltpu.async_copy(scales_hbm, sc_v, sem)
    c3 = pltpu.async_copy(idx_hbm.at[pl.ds(base, _CHUNK)], idx_v, sem)
    c4 = pltpu.async_copy(x_hbm.at[pl.ds(base, _CHUNK)], x_v, sem)
    c1.wait()
    c2.wait()
    c3.wait()
    c4.wait()

    @plsc.parallel_loop(0, _CHUNK, step=_L, unroll=8)
    def _step(o):
        iv = idx_v[pl.ds(o, _L)]
        xv = x_v[pl.ds(o, _L)]
        sh = plsc.load_gather(sh_v, [iv])
        sc = plsc.load_gather(sc_v, [iv])
        o_v[pl.ds(o, _L)] = sh + sc * xv

    pltpu.sync_copy(o_v, out_hbm.at[pl.ds(base, _CHUNK)])


@jax.jit
def _sc_shift(x, idx, shifts, scales):
    mesh = plsc.VectorSubcoreMesh(core_axis_name="c", subcore_axis_name="s")
    fn = pl.kernel(
        _sc_body,
        out_type=jax.ShapeDtypeStruct((_N,), jnp.float32),
        mesh=mesh,
        scratch_types=[
            pltpu.VMEM((_CHUNK,), jnp.int32),
            pltpu.VMEM((_CHUNK,), jnp.float32),
            pltpu.VMEM((_CHUNK,), jnp.float32),
            pltpu.VMEM((_S,), jnp.float32),
            pltpu.VMEM((_S,), jnp.float32),
            pltpu.SemaphoreType.DMA,
        ],
        compiler_params=pltpu.CompilerParams(needs_layout_passes=False),
    )
    return fn(x, idx, shifts, scales)


def kernel(x, species_idx, shifts, scales):
    out = _sc_shift(x.reshape(-1), species_idx.astype(jnp.int32),
                    shifts, scales)
    return out.reshape(_N, 1)


# speedup vs baseline: 4.1425x; 1.0031x over previous
import jax
import jax.numpy as jnp
from jax import lax
from jax.experimental import pallas as pl
from jax.experimental.pallas import tpu as pltpu
from jax.experimental.pallas import tpu_sc as plsc

_N = 100000
_S = 64
_L = 16
_NC = 2
_NS = 16
_NW = _NC * _NS
_CHUNK = 3136


def _sc_body(x_hbm, idx_hbm, shifts_hbm, scales_hbm, out_hbm,
             idx_v, x_v, o_v, sh_v, sc_v, sem):
    wid = lax.axis_index("s") * _NC + lax.axis_index("c")
    base = jnp.minimum(wid * _CHUNK, _N - _CHUNK)

    c1 = pltpu.async_copy(shifts_hbm, sh_v, sem)
    c2 = pltpu.async_copy(scales_hbm, sc_v, sem)
    c3 = pltpu.async_copy(idx_hbm.at[pl.ds(base, _CHUNK)], idx_v, sem)
    c4 = pltpu.async_copy(x_hbm.at[pl.ds(base, _CHUNK)], x_v, sem)
    c1.wait()
    c2.wait()
    c3.wait()
    c4.wait()

    @plsc.parallel_loop(0, _CHUNK, step=_L, unroll=2)
    def _step(o):
        iv = idx_v[pl.ds(o, _L)]
        xv = x_v[pl.ds(o, _L)]
        sh = plsc.load_gather(sh_v, [iv])
        sc = plsc.load_gather(sc_v, [iv])
        o_v[pl.ds(o, _L)] = sh + sc * xv

    pltpu.sync_copy(o_v, out_hbm.at[pl.ds(base, _CHUNK)])


@jax.jit
def _sc_shift(x, idx, shifts, scales):
    mesh = plsc.VectorSubcoreMesh(core_axis_name="c", subcore_axis_name="s")
    fn = pl.kernel(
        _sc_body,
        out_type=jax.ShapeDtypeStruct((_N,), jnp.float32),
        mesh=mesh,
        scratch_types=[
            pltpu.VMEM((_CHUNK,), jnp.int32),
            pltpu.VMEM((_CHUNK,), jnp.float32),
            pltpu.VMEM((_CHUNK,), jnp.float32),
            pltpu.VMEM((_S,), jnp.float32),
            pltpu.VMEM((_S,), jnp.float32),
            pltpu.SemaphoreType.DMA,
        ],
        compiler_params=pltpu.CompilerParams(needs_layout_passes=False),
    )
    return fn(x, idx, shifts, scales)


def kernel(x, species_idx, shifts, scales):
    out = _sc_shift(x.reshape(-1), species_idx.astype(jnp.int32),
                    shifts, scales)
    return out.reshape(_N, 1)
